# baseline (speedup 1.0000x reference)
import functools

import jax
import jax.numpy as jnp
from jax import lax
from jax.experimental import pallas as pl
from jax.experimental.pallas import tpu as pltpu
from jax.experimental.pallas import tpu_sc as plsc

_BATCH = 4
_ROWS = 2048
_FEAT = 1024
_NUM_CORES = 2
_NUM_SUBCORES = 16
_NUM_WORKERS = _NUM_CORES * _NUM_SUBCORES
_ROWS_PER_WORKER = _ROWS // _NUM_WORKERS


@jax.jit
def _broadcast_table(w):
    mesh = plsc.VectorSubcoreMesh(core_axis_name="c", subcore_axis_name="s")

    @functools.partial(
        pl.kernel,
        mesh=mesh,
        out_type=jax.ShapeDtypeStruct((_BATCH, _ROWS, _FEAT), jnp.float32),
        scratch_types=[
            pltpu.SemaphoreType.DMA,
        ],
    )
    def k(w_hbm, out_hbm, sem):
        wid = lax.axis_index("s") * _NUM_CORES + lax.axis_index("c")
        base = wid * _ROWS_PER_WORKER
        copies = [
            pltpu.async_copy(
                w_hbm.at[pl.ds(base, _ROWS_PER_WORKER)],
                out_hbm.at[b, pl.ds(base, _ROWS_PER_WORKER)],
                sem,
            )
            for b in range(_BATCH)
        ]
        for c in copies:
            c.wait()

    return k(w)


def kernel(seq_in_embeds, W):
    del seq_in_embeds
    return _broadcast_table(W)


# speedup vs baseline: 31.3645x; 31.3645x over previous
import functools

import jax
import jax.numpy as jnp
from jax import lax
from jax.experimental import pallas as pl
from jax.experimental.pallas import tpu as pltpu
from jax.experimental.pallas import tpu_sc as plsc

_BATCH = 4
_ROWS = 2048
_FEAT = 1024
_NUM_CORES = 2
_NUM_SUBCORES = 16
_NUM_WORKERS = _NUM_CORES * _NUM_SUBCORES
_ROWS_PER_WORKER = _ROWS // _NUM_WORKERS
_CHUNKS = 4
_CHUNK_ROWS = _ROWS_PER_WORKER // _CHUNKS


@jax.jit
def _broadcast_table(w):
    mesh = plsc.VectorSubcoreMesh(core_axis_name="c", subcore_axis_name="s")

    @functools.partial(
        pl.kernel,
        mesh=mesh,
        out_type=jax.ShapeDtypeStruct((_BATCH, _ROWS, _FEAT), jnp.float32),
        scratch_types=(
            [pltpu.VMEM((_CHUNK_ROWS, _FEAT), jnp.float32)] * _CHUNKS
            + [pltpu.SemaphoreType.DMA] * _CHUNKS
            + [pltpu.SemaphoreType.DMA]
        ),
    )
    def k(w_hbm, out_hbm, *scratch):
        bufs = scratch[:_CHUNKS]
        load_sems = scratch[_CHUNKS : 2 * _CHUNKS]
        store_sem = scratch[2 * _CHUNKS]
        wid = lax.axis_index("s") * _NUM_CORES + lax.axis_index("c")
        base = wid * _ROWS_PER_WORKER
        loads = [
            pltpu.async_copy(
                w_hbm.at[pl.ds(base + c * _CHUNK_ROWS, _CHUNK_ROWS)],
                bufs[c],
                load_sems[c],
            )
            for c in range(_CHUNKS)
        ]
        stores = []
        for c in range(_CHUNKS):
            loads[c].wait()
            for b in range(_BATCH):
                stores.append(
                    pltpu.async_copy(
                        bufs[c],
                        out_hbm.at[b, pl.ds(base + c * _CHUNK_ROWS, _CHUNK_ROWS)],
                        store_sem,
                    )
                )
        for s in stores:
            s.wait()

    return k(w)


def kernel(seq_in_embeds, W):
    del seq_in_embeds
    return _broadcast_table(W)


# speedup vs baseline: 31.9136x; 1.0175x over previous
import functools

import jax
import jax.numpy as jnp
from jax import lax
from jax.experimental import pallas as pl
from jax.experimental.pallas import tpu as pltpu
from jax.experimental.pallas import tpu_sc as plsc

_NUM_CORES = 2
_NUM_SUBCORES = 16
_NUM_WORKERS = _NUM_CORES * _NUM_SUBCORES


@functools.lru_cache(maxsize=None)
def _make_broadcast(batch, seq_len, feat):
    rows_per_worker = seq_len // _NUM_WORKERS
    tail_rows = seq_len - rows_per_worker * _NUM_WORKERS
    buf_rows = max(rows_per_worker, tail_rows, 1)
    mesh = plsc.VectorSubcoreMesh(core_axis_name="c", subcore_axis_name="s")

    @jax.jit
    @functools.partial(
        pl.kernel,
        mesh=mesh,
        out_type=jax.ShapeDtypeStruct((batch, seq_len, feat), jnp.float32),
        scratch_types=[
            pltpu.VMEM((buf_rows, feat), jnp.float32),
            pltpu.SemaphoreType.DMA,
        ],
    )
    def k(w_hbm, out_hbm, buf, sem):
        wid = lax.axis_index("s") * _NUM_CORES + lax.axis_index("c")

        def emit(base, nrows):
            pltpu.sync_copy(w_hbm.at[pl.ds(base, nrows)], buf.at[pl.ds(0, nrows)])
            stores = [
                pltpu.async_copy(
                    buf.at[pl.ds(0, nrows)],
                    out_hbm.at[b, pl.ds(base, nrows)],
                    sem,
                )
                for b in range(batch)
            ]
            for s in stores:
                s.wait()

        if rows_per_worker > 0:
            emit(wid * rows_per_worker, rows_per_worker)
        if tail_rows > 0:
            @pl.when(wid == 0)
            def _():
                emit(_NUM_WORKERS * rows_per_worker, tail_rows)

    return k


def kernel(seq_in_embeds, W):
    batch, seq_len = seq_in_embeds.shape[0], seq_in_embeds.shape[1]
    return _make_broadcast(batch, seq_len, W.shape[1])(W)
